# baseline (speedup 1.0000x reference)
import jax
import jax.numpy as jnp
from jax import lax
from jax.experimental import pallas as pl
from jax.experimental.pallas import tpu as pltpu
from jax.experimental.pallas import tpu_sc as plsc

N0 = 100000
NPAD = 100352
NL = NPAD // 8
RPT_N = NPAD // 16
E0 = 3200000
SUPE = 512
NSUP = 196
EPT = SUPE * NSUP
EPAD = 32 * EPT
SUPD = 1024
NSUPD = EPT // SUPD

_MESH = plsc.VectorSubcoreMesh(core_axis_name="c", subcore_axis_name="s",
                               num_cores=2, num_subcores=16)


def _deg_body(dst1, degp, idxd0, idxd1, ones_v, zbuf, accd, semd0, semd1):
    cid = lax.axis_index("c")
    sid = lax.axis_index("s")
    wid = sid * 2 + cid
    zv = jnp.zeros((16,), jnp.float32)
    ov = jnp.ones((16,), jnp.float32)

    def fill_z(k, carry):
        zbuf[pl.ds(k * 16, 16)] = zv
        return carry
    lax.fori_loop(0, RPT_N // 16, fill_z, 0)

    def fill_o(i, carry):
        ones_v[pl.ds(i * 16, 16)] = ov
        return carry
    lax.fori_loop(0, SUPD // 16, fill_o, 0)

    nb = sid * RPT_N
    pltpu.sync_copy(zbuf, accd.at[pl.ds(nb, RPT_N)])
    plsc.subcore_barrier()

    eb = wid * EPT

    idxd = (idxd0, idxd1)
    pltpu.sync_copy(dst1.at[pl.ds(eb, SUPD)], idxd[0])

    def dpair(s, carry):
        for b in range(2):
            c = s * 2 + b
            p = b
            q = 1 - b
            sc = pltpu.async_copy(ones_v, accd.at[idxd[p]],
                                  semd0 if b == 0 else semd1, add=True)
            cn = lax.rem(c + 1, NSUPD)
            pltpu.sync_copy(dst1.at[pl.ds(eb + cn * SUPD, SUPD)], idxd[q])
            sc.wait()
        return carry
    lax.fori_loop(0, NSUPD // 2, dpair, 0)
    plsc.subcore_barrier()
    pltpu.sync_copy(accd.at[pl.ds(nb, RPT_N)], degp.at[cid, pl.ds(nb, RPT_N)])


_deg_call = pl.kernel(
    _deg_body,
    out_type=jax.ShapeDtypeStruct((2, NPAD), jnp.float32),
    mesh=_MESH,
    scratch_types=[
        pltpu.VMEM((SUPD,), jnp.int32),
        pltpu.VMEM((SUPD,), jnp.int32),
        pltpu.VMEM((SUPD,), jnp.float32),
        pltpu.VMEM((RPT_N,), jnp.float32),
        pltpu.VMEM_SHARED((NPAD,), jnp.float32),
        pltpu.SemaphoreType.DMA,
        pltpu.SemaphoreType.DMA,
    ],
    compiler_params=pltpu.CompilerParams(use_tc_tiling_on_sc=False),
)


def _agg_body(table, src1, dst1, aggp,
              idxs0, idxd0, rows0, idxs1, idxd1, rows1, zbuf, acc,
              semg0, semg1, sems0, sems1):
    cid = lax.axis_index("c")
    sid = lax.axis_index("s")
    wid = sid * 2 + cid
    zv = jnp.zeros((16,), jnp.float32)

    idxs = (idxs0, idxs1)
    idxd = (idxd0, idxd1)
    rows = (rows0, rows1)
    semg = (semg0, semg1)

    def fill_z(i, carry):
        zbuf[i, :] = zv
        return carry
    lax.fori_loop(0, 128, fill_z, 0)

    nb = sid * RPT_N

    def zcopy(j, carry):
        pltpu.sync_copy(zbuf, acc.at[pl.ds(nb + j * 128, 128)])
        return carry
    lax.fori_loop(0, RPT_N // 128, zcopy, 0)
    plsc.subcore_barrier()

    eb = wid * EPT

    def stage(c, p):
        rb = eb + c * SUPE
        pltpu.sync_copy(src1.at[pl.ds(rb, SUPE)], idxs[p])
        pltpu.sync_copy(dst1.at[pl.ds(rb, SUPE)], idxd[p])
        return pltpu.async_copy(table.at[idxs[p]], rows[p], semg[p])

    stage(0, 0)

    def pair(s, carry):
        for b in range(2):
            c = s * 2 + b
            p = b
            q = 1 - b
            pltpu.make_async_copy(table.at[idxs[p]], rows[p], semg[p]).wait()
            sc = pltpu.async_copy(rows[p], acc.at[idxd[p]],
                                  sems0 if b == 0 else sems1, add=True)
            stage(lax.rem(c + 1, NSUP), q)
            sc.wait()
        return carry
    lax.fori_loop(0, NSUP // 2, pair, 0)
    pltpu.make_async_copy(table.at[idxs[0]], rows[0], semg[0]).wait()
    plsc.subcore_barrier()
    pltpu.sync_copy(acc.at[pl.ds(nb, RPT_N)], aggp.at[cid, pl.ds(nb, RPT_N)])


_agg_call = pl.kernel(
    _agg_body,
    out_type=jax.ShapeDtypeStruct((2, NPAD, 16), jnp.float32),
    mesh=_MESH,
    scratch_types=[
        pltpu.VMEM((SUPE,), jnp.int32),
        pltpu.VMEM((SUPE,), jnp.int32),
        pltpu.VMEM((SUPE, 16), jnp.float32),
        pltpu.VMEM((SUPE,), jnp.int32),
        pltpu.VMEM((SUPE,), jnp.int32),
        pltpu.VMEM((SUPE, 16), jnp.float32),
        pltpu.VMEM((128, 16), jnp.float32),
        pltpu.VMEM_SHARED((NPAD, 16), jnp.float32),
        pltpu.SemaphoreType.DMA,
        pltpu.SemaphoreType.DMA,
        pltpu.SemaphoreType.DMA,
        pltpu.SemaphoreType.DMA,
    ],
    compiler_params=pltpu.CompilerParams(use_tc_tiling_on_sc=False),
)


BR = 2048
BL = BR // 8
GRID = NPAD // BR


def _lin_body(xv, bw, xl_lin):
    xl_lin[:] = jnp.dot(xv[:], bw[:], preferred_element_type=jnp.float32)


_lin_call = pl.pallas_call(
    _lin_body,
    grid=(GRID,),
    in_specs=[
        pl.BlockSpec((BL, 40), lambda i: (i, 0)),
        pl.BlockSpec((40, 128), lambda i: (0, 0)),
    ],
    out_specs=pl.BlockSpec((BL, 128), lambda i: (i, 0)),
    out_shape=jax.ShapeDtypeStruct((NL, 128), jnp.float32),
)


def _scale_body(d0, d1, xl, e8, y1, dinv_e):
    di = lax.rsqrt(1.0 + d0[:] + d1[:])
    de = jnp.dot(di, e8[:], preferred_element_type=jnp.float32)
    dinv_e[:] = de
    y1[:] = xl[:] * de


_scale_call = pl.pallas_call(
    _scale_body,
    grid=(GRID,),
    in_specs=[
        pl.BlockSpec((BL, 8), lambda i: (i, 0)),
        pl.BlockSpec((BL, 8), lambda i: (i + GRID, 0)),
        pl.BlockSpec((BL, 128), lambda i: (i, 0)),
        pl.BlockSpec((8, 128), lambda i: (0, 0)),
    ],
    out_specs=[
        pl.BlockSpec((BL, 128), lambda i: (i, 0)),
        pl.BlockSpec((BL, 128), lambda i: (i, 0)),
    ],
    out_shape=[
        jax.ShapeDtypeStruct((NL, 128), jnp.float32),
        jax.ShapeDtypeStruct((NL, 128), jnp.float32),
    ],
)


def _mid_body(a0, a1, y1, de, b1e, z2):
    i = pl.program_id(0)
    h = de[:] * (a0[:] + a1[:] + y1[:]) + b1e[:][None, :]
    h = jnp.maximum(h, 0.0)
    rows = i * BL + lax.broadcasted_iota(jnp.int32, (BL, 1), 0)
    z2[:] = jnp.where(rows < N0 // 8, de[:] * h, 0.0)


_mid_call = pl.pallas_call(
    _mid_body,
    grid=(GRID,),
    in_specs=[
        pl.BlockSpec((BL, 128), lambda i: (i, 0)),
        pl.BlockSpec((BL, 128), lambda i: (i + GRID, 0)),
        pl.BlockSpec((BL, 128), lambda i: (i, 0)),
        pl.BlockSpec((BL, 128), lambda i: (i, 0)),
        pl.BlockSpec((128,), lambda i: (0,)),
    ],
    out_specs=pl.BlockSpec((BL, 128), lambda i: (i, 0)),
    out_shape=jax.ShapeDtypeStruct((NL, 128), jnp.float32),
)


def _fin_body(a0, a1, z2, de, w2b, b2e, swp, o):
    g = de[:] * (a0[:] + a1[:] + z2[:])
    t = jnp.dot(g, w2b[:], preferred_element_type=jnp.float32)
    t = t + b2e[:][None, :]
    tsw = jnp.dot(t, swp[:], preferred_element_type=jnp.float32)
    m = jnp.maximum(t, tsw)
    s = t - m
    es = jnp.exp(s)
    essw = jnp.dot(es, swp[:], preferred_element_type=jnp.float32)
    o[:] = s - jnp.log(es + essw)


_fin_call = pl.pallas_call(
    _fin_body,
    grid=(GRID,),
    in_specs=[
        pl.BlockSpec((BL, 128), lambda i: (i, 0)),
        pl.BlockSpec((BL, 128), lambda i: (i + GRID, 0)),
        pl.BlockSpec((BL, 128), lambda i: (i, 0)),
        pl.BlockSpec((BL, 128), lambda i: (i, 0)),
        pl.BlockSpec((128, 16), lambda i: (0, 0)),
        pl.BlockSpec((16,), lambda i: (0,)),
        pl.BlockSpec((16, 16), lambda i: (0, 0)),
    ],
    out_specs=pl.BlockSpec((BL, 16), lambda i: (i, 0)),
    out_shape=jax.ShapeDtypeStruct((NL, 16), jnp.float32),
)


def kernel(x, edge_index, W1, b1, W2, b2):
    pad_e = EPAD - E0
    pad_idx = N0 + jnp.arange(pad_e, dtype=jnp.int32) % (NPAD - N0)
    src1 = jnp.concatenate([edge_index[0], pad_idx])
    dst1 = jnp.concatenate([edge_index[1], pad_idx])
    xv = jnp.zeros((NL, 40), jnp.float32).at[:N0 // 8].set(
        x.astype(jnp.float32).reshape(N0 // 8, 40))
    eye8 = jnp.eye(8, dtype=jnp.float32)
    bw = jnp.kron(eye8, W1)
    e8 = jnp.kron(eye8, jnp.ones((1, 16), jnp.float32))
    w2b = jnp.kron(eye8, W2)
    swp = jnp.kron(eye8, jnp.array([[0., 1.], [1., 0.]],
                                   jnp.float32))
    b1e = jnp.tile(b1, 8)
    b2e = jnp.tile(b2, 8)

    xl = _lin_call(xv, bw)
    degp8 = _deg_call(dst1).reshape(2 * NL, 8)
    y1, dinv_e = _scale_call(degp8, degp8, xl, e8)
    a1 = _agg_call(y1.reshape(NPAD, 16), src1, dst1).reshape(2 * NL, 128)
    z2 = _mid_call(a1, a1, y1, dinv_e, b1e)
    a2 = _agg_call(z2.reshape(NPAD, 16), src1, dst1).reshape(2 * NL, 128)
    o16 = _fin_call(a2, a2, z2, dinv_e, w2b, b2e, swp)
    return o16.reshape(NPAD, 2)[:N0]


# speedup vs baseline: 1.3323x; 1.3323x over previous
import jax
import jax.numpy as jnp
from jax import lax
from jax.experimental import pallas as pl
from jax.experimental.pallas import tpu as pltpu
from jax.experimental.pallas import tpu_sc as plsc

N0 = 100000
NPAD = 100352
NL = NPAD // 8
RPT_N = NPAD // 16
E0 = 3200000
SUPE = 1024
NSUP = 98
EPT = SUPE * NSUP
EPAD = 32 * EPT
SUPD = 1024
NSUPD = EPT // SUPD

_MESH = plsc.VectorSubcoreMesh(core_axis_name="c", subcore_axis_name="s",
                               num_cores=2, num_subcores=16)


def _deg_body(dst1, degp, idxd0, idxd1, ones_v, zbuf, accd, semd0, semd1):
    cid = lax.axis_index("c")
    sid = lax.axis_index("s")
    wid = sid * 2 + cid
    zv = jnp.zeros((16,), jnp.float32)
    ov = jnp.ones((16,), jnp.float32)

    def fill_z(k, carry):
        zbuf[pl.ds(k * 16, 16)] = zv
        return carry
    lax.fori_loop(0, RPT_N // 16, fill_z, 0)

    def fill_o(i, carry):
        ones_v[pl.ds(i * 16, 16)] = ov
        return carry
    lax.fori_loop(0, SUPD // 16, fill_o, 0)

    nb = sid * RPT_N
    pltpu.sync_copy(zbuf, accd.at[pl.ds(nb, RPT_N)])
    plsc.subcore_barrier()

    eb = wid * EPT

    idxd = (idxd0, idxd1)
    pltpu.sync_copy(dst1.at[pl.ds(eb, SUPD)], idxd[0])

    def dpair(s, carry):
        for b in range(2):
            c = s * 2 + b
            p = b
            q = 1 - b
            sc = pltpu.async_copy(ones_v, accd.at[idxd[p]],
                                  semd0 if b == 0 else semd1, add=True)
            cn = lax.rem(c + 1, NSUPD)
            pltpu.sync_copy(dst1.at[pl.ds(eb + cn * SUPD, SUPD)], idxd[q])
            sc.wait()
        return carry
    lax.fori_loop(0, NSUPD // 2, dpair, 0)
    plsc.subcore_barrier()
    pltpu.sync_copy(accd.at[pl.ds(nb, RPT_N)], degp.at[cid, pl.ds(nb, RPT_N)])


_deg_call = pl.kernel(
    _deg_body,
    out_type=jax.ShapeDtypeStruct((2, NPAD), jnp.float32),
    mesh=_MESH,
    scratch_types=[
        pltpu.VMEM((SUPD,), jnp.int32),
        pltpu.VMEM((SUPD,), jnp.int32),
        pltpu.VMEM((SUPD,), jnp.float32),
        pltpu.VMEM((RPT_N,), jnp.float32),
        pltpu.VMEM_SHARED((NPAD,), jnp.float32),
        pltpu.SemaphoreType.DMA,
        pltpu.SemaphoreType.DMA,
    ],
    compiler_params=pltpu.CompilerParams(use_tc_tiling_on_sc=False),
)


def _agg_body(table, src1, dst1, aggp,
              idxs0, idxd0, rows0, idxs1, idxd1, zbuf, acc,
              semg0, semg1, sems0):
    cid = lax.axis_index("c")
    sid = lax.axis_index("s")
    wid = sid * 2 + cid
    zv = jnp.zeros((16,), jnp.float32)

    idxs = (idxs0, idxs1)
    idxd = (idxd0, idxd1)

    def fill_z(i, carry):
        zbuf[i, :] = zv
        return carry
    lax.fori_loop(0, 128, fill_z, 0)

    nb = sid * RPT_N
    zcps = [pltpu.async_copy(zbuf, acc.at[pl.ds(nb + j * 128, 128)], semg1)
            for j in range(RPT_N // 128)]
    for cp in zcps:
        cp.wait()
    plsc.subcore_barrier()

    eb = wid * EPT

    def load_idx(c, p):
        rb = eb + c * SUPE
        pltpu.sync_copy(src1.at[pl.ds(rb, SUPE)], idxs[p])
        pltpu.sync_copy(dst1.at[pl.ds(rb, SUPE)], idxd[p])

    load_idx(0, 0)

    def pair(s, carry):
        for b in range(2):
            c = s * 2 + b
            p = b
            q = 1 - b
            pltpu.async_copy(table.at[idxs[p]], rows0, semg0).wait()
            sc = pltpu.async_copy(rows0, acc.at[idxd[p]], sems0, add=True)
            load_idx(lax.rem(c + 1, NSUP), q)
            sc.wait()
        return carry
    lax.fori_loop(0, NSUP // 2, pair, 0)
    plsc.subcore_barrier()
    pltpu.sync_copy(acc.at[pl.ds(nb, RPT_N)], aggp.at[cid, pl.ds(nb, RPT_N)])


_agg_call = pl.kernel(
    _agg_body,
    out_type=jax.ShapeDtypeStruct((2, NPAD, 16), jnp.float32),
    mesh=_MESH,
    scratch_types=[
        pltpu.VMEM((SUPE,), jnp.int32),
        pltpu.VMEM((SUPE,), jnp.int32),
        pltpu.VMEM((SUPE, 16), jnp.float32),
        pltpu.VMEM((SUPE,), jnp.int32),
        pltpu.VMEM((SUPE,), jnp.int32),
        pltpu.VMEM((128, 16), jnp.float32),
        pltpu.VMEM_SHARED((NPAD, 16), jnp.float32),
        pltpu.SemaphoreType.DMA,
        pltpu.SemaphoreType.DMA,
        pltpu.SemaphoreType.DMA,
    ],
    compiler_params=pltpu.CompilerParams(use_tc_tiling_on_sc=False),
)


BR = 2048
BL = BR // 8
GRID = NPAD // BR


def _lin_body(xv, bw, xl_lin):
    xl_lin[:] = jnp.dot(xv[:], bw[:], preferred_element_type=jnp.float32)


_lin_call = pl.pallas_call(
    _lin_body,
    grid=(GRID,),
    in_specs=[
        pl.BlockSpec((BL, 40), lambda i: (i, 0)),
        pl.BlockSpec((40, 128), lambda i: (0, 0)),
    ],
    out_specs=pl.BlockSpec((BL, 128), lambda i: (i, 0)),
    out_shape=jax.ShapeDtypeStruct((NL, 128), jnp.float32),
)


def _scale_body(d0, d1, xl, e8, y1, dinv_e):
    di = lax.rsqrt(1.0 + d0[:] + d1[:])
    de = jnp.dot(di, e8[:], preferred_element_type=jnp.float32)
    dinv_e[:] = de
    y1[:] = xl[:] * de


_scale_call = pl.pallas_call(
    _scale_body,
    grid=(GRID,),
    in_specs=[
        pl.BlockSpec((BL, 8), lambda i: (i, 0)),
        pl.BlockSpec((BL, 8), lambda i: (i + GRID, 0)),
        pl.BlockSpec((BL, 128), lambda i: (i, 0)),
        pl.BlockSpec((8, 128), lambda i: (0, 0)),
    ],
    out_specs=[
        pl.BlockSpec((BL, 128), lambda i: (i, 0)),
        pl.BlockSpec((BL, 128), lambda i: (i, 0)),
    ],
    out_shape=[
        jax.ShapeDtypeStruct((NL, 128), jnp.float32),
        jax.ShapeDtypeStruct((NL, 128), jnp.float32),
    ],
)


def _mid_body(a0, a1, y1, de, b1e, z2):
    i = pl.program_id(0)
    h = de[:] * (a0[:] + a1[:] + y1[:]) + b1e[:][None, :]
    h = jnp.maximum(h, 0.0)
    rows = i * BL + lax.broadcasted_iota(jnp.int32, (BL, 1), 0)
    z2[:] = jnp.where(rows < N0 // 8, de[:] * h, 0.0)


_mid_call = pl.pallas_call(
    _mid_body,
    grid=(GRID,),
    in_specs=[
        pl.BlockSpec((BL, 128), lambda i: (i, 0)),
        pl.BlockSpec((BL, 128), lambda i: (i + GRID, 0)),
        pl.BlockSpec((BL, 128), lambda i: (i, 0)),
        pl.BlockSpec((BL, 128), lambda i: (i, 0)),
        pl.BlockSpec((128,), lambda i: (0,)),
    ],
    out_specs=pl.BlockSpec((BL, 128), lambda i: (i, 0)),
    out_shape=jax.ShapeDtypeStruct((NL, 128), jnp.float32),
)


def _fin_body(a0, a1, z2, de, w2b, b2e, swp, o):
    g = de[:] * (a0[:] + a1[:] + z2[:])
    t = jnp.dot(g, w2b[:], preferred_element_type=jnp.float32)
    t = t + b2e[:][None, :]
    tsw = jnp.dot(t, swp[:], preferred_element_type=jnp.float32)
    m = jnp.maximum(t, tsw)
    s = t - m
    es = jnp.exp(s)
    essw = jnp.dot(es, swp[:], preferred_element_type=jnp.float32)
    o[:] = s - jnp.log(es + essw)


_fin_call = pl.pallas_call(
    _fin_body,
    grid=(GRID,),
    in_specs=[
        pl.BlockSpec((BL, 128), lambda i: (i, 0)),
        pl.BlockSpec((BL, 128), lambda i: (i + GRID, 0)),
        pl.BlockSpec((BL, 128), lambda i: (i, 0)),
        pl.BlockSpec((BL, 128), lambda i: (i, 0)),
        pl.BlockSpec((128, 16), lambda i: (0, 0)),
        pl.BlockSpec((16,), lambda i: (0,)),
        pl.BlockSpec((16, 16), lambda i: (0, 0)),
    ],
    out_specs=pl.BlockSpec((BL, 16), lambda i: (i, 0)),
    out_shape=jax.ShapeDtypeStruct((NL, 16), jnp.float32),
)


def kernel(x, edge_index, W1, b1, W2, b2):
    pad_e = EPAD - E0
    pad_idx = N0 + jnp.arange(pad_e, dtype=jnp.int32) % (NPAD - N0)
    src1 = jnp.concatenate([edge_index[0], pad_idx])
    dst1 = jnp.concatenate([edge_index[1], pad_idx])
    xv = jnp.zeros((NL, 40), jnp.float32).at[:N0 // 8].set(
        x.astype(jnp.float32).reshape(N0 // 8, 40))
    eye8 = jnp.eye(8, dtype=jnp.float32)
    bw = jnp.kron(eye8, W1)
    e8 = jnp.kron(eye8, jnp.ones((1, 16), jnp.float32))
    w2b = jnp.kron(eye8, W2)
    swp = jnp.kron(eye8, jnp.array([[0., 1.], [1., 0.]],
                                   jnp.float32))
    b1e = jnp.tile(b1, 8)
    b2e = jnp.tile(b2, 8)

    xl = _lin_call(xv, bw)
    degp8 = _deg_call(dst1).reshape(2 * NL, 8)
    y1, dinv_e = _scale_call(degp8, degp8, xl, e8)
    a1 = _agg_call(y1.reshape(NPAD, 16), src1, dst1).reshape(2 * NL, 128)
    z2 = _mid_call(a1, a1, y1, dinv_e, b1e)
    a2 = _agg_call(z2.reshape(NPAD, 16), src1, dst1).reshape(2 * NL, 128)
    o16 = _fin_call(a2, a2, z2, dinv_e, w2b, b2e, swp)
    return o16.reshape(NPAD, 2)[:N0]


# speedup vs baseline: 1.3643x; 1.0240x over previous
import jax
import jax.numpy as jnp
from jax import lax
from jax.experimental import pallas as pl
from jax.experimental.pallas import tpu as pltpu
from jax.experimental.pallas import tpu_sc as plsc

N0 = 100000
NPAD = 100352
NL = NPAD // 8
RPT_N = NPAD // 16
E0 = 3200000
EPT = E0 // 32
SUPE = 1024
NFULL = 97
TAIL = EPT - NFULL * SUPE

_MESH = plsc.VectorSubcoreMesh(core_axis_name="c", subcore_axis_name="s",
                               num_cores=2, num_subcores=16)


def _deg_body(dst1, degp, idxd0, idxd1, idxt, ones_v, zbuf, accd,
              semd0, semd1):
    cid = lax.axis_index("c")
    sid = lax.axis_index("s")
    wid = sid * 2 + cid
    zv = jnp.zeros((16,), jnp.float32)
    ov = jnp.ones((16,), jnp.float32)

    def fill_z(k, carry):
        zbuf[pl.ds(k * 16, 16)] = zv
        return carry
    lax.fori_loop(0, RPT_N // 16, fill_z, 0)

    def fill_o(i, carry):
        ones_v[pl.ds(i * 16, 16)] = ov
        return carry
    lax.fori_loop(0, SUPE // 16, fill_o, 0)

    nb = sid * RPT_N
    pltpu.sync_copy(zbuf, accd.at[pl.ds(nb, RPT_N)])
    plsc.subcore_barrier()

    eb = wid * EPT

    idxd = (idxd0, idxd1)
    pltpu.sync_copy(dst1.at[pl.ds(eb, SUPE)], idxd[0])

    def dpair(s, carry):
        for b in range(2):
            c = s * 2 + b
            p = b
            q = 1 - b
            sc = pltpu.async_copy(ones_v, accd.at[idxd[p]],
                                  semd0 if b == 0 else semd1, add=True)
            pltpu.sync_copy(dst1.at[pl.ds(eb + (c + 1) * SUPE, SUPE)],
                            idxd[q])
            sc.wait()
        return carry
    lax.fori_loop(0, (NFULL - 1) // 2, dpair, 0)
    sc = pltpu.async_copy(ones_v, accd.at[idxd[0]], semd0, add=True)
    pltpu.sync_copy(dst1.at[pl.ds(eb + NFULL * SUPE, TAIL)], idxt)
    sc.wait()
    pltpu.async_copy(ones_v.at[pl.ds(0, TAIL)], accd.at[idxt], semd1,
                     add=True).wait()
    plsc.subcore_barrier()
    pltpu.sync_copy(accd.at[pl.ds(nb, RPT_N)], degp.at[cid, pl.ds(nb, RPT_N)])


_deg_call = pl.kernel(
    _deg_body,
    out_type=jax.ShapeDtypeStruct((2, NPAD), jnp.float32),
    mesh=_MESH,
    scratch_types=[
        pltpu.VMEM((SUPE,), jnp.int32),
        pltpu.VMEM((SUPE,), jnp.int32),
        pltpu.VMEM((TAIL,), jnp.int32),
        pltpu.VMEM((SUPE,), jnp.float32),
        pltpu.VMEM((RPT_N,), jnp.float32),
        pltpu.VMEM_SHARED((NPAD,), jnp.float32),
        pltpu.SemaphoreType.DMA,
        pltpu.SemaphoreType.DMA,
    ],
    compiler_params=pltpu.CompilerParams(use_tc_tiling_on_sc=False),
)


def _agg_body(table, src1, dst1, aggp,
              idxs0, idxd0, rows0, idxs1, idxd1, idxst, idxdt, zbuf, acc,
              semg0, semg1, sems0):
    cid = lax.axis_index("c")
    sid = lax.axis_index("s")
    wid = sid * 2 + cid
    zv = jnp.zeros((16,), jnp.float32)

    idxs = (idxs0, idxs1)
    idxd = (idxd0, idxd1)

    def fill_z(i, carry):
        zbuf[i, :] = zv
        return carry
    lax.fori_loop(0, 128, fill_z, 0)

    nb = sid * RPT_N
    zcps = [pltpu.async_copy(zbuf, acc.at[pl.ds(nb + j * 128, 128)], semg1)
            for j in range(RPT_N // 128)]
    for cp in zcps:
        cp.wait()
    plsc.subcore_barrier()

    eb = wid * EPT

    def load_idx(c, p):
        rb = eb + c * SUPE
        pltpu.sync_copy(src1.at[pl.ds(rb, SUPE)], idxs[p])
        pltpu.sync_copy(dst1.at[pl.ds(rb, SUPE)], idxd[p])

    load_idx(0, 0)

    def pair(s, carry):
        for b in range(2):
            c = s * 2 + b
            p = b
            q = 1 - b
            pltpu.async_copy(table.at[idxs[p]], rows0, semg0).wait()
            sc = pltpu.async_copy(rows0, acc.at[idxd[p]], sems0, add=True)
            load_idx(c + 1, q)
            sc.wait()
        return carry
    lax.fori_loop(0, (NFULL - 1) // 2, pair, 0)
    pltpu.async_copy(table.at[idxs[0]], rows0, semg0).wait()
    sc = pltpu.async_copy(rows0, acc.at[idxd[0]], sems0, add=True)
    tb = eb + NFULL * SUPE
    pltpu.sync_copy(src1.at[pl.ds(tb, TAIL)], idxst)
    pltpu.sync_copy(dst1.at[pl.ds(tb, TAIL)], idxdt)
    sc.wait()
    pltpu.async_copy(table.at[idxst], rows0.at[pl.ds(0, TAIL)], semg0).wait()
    pltpu.async_copy(rows0.at[pl.ds(0, TAIL)], acc.at[idxdt], sems0,
                     add=True).wait()
    plsc.subcore_barrier()
    pltpu.sync_copy(acc.at[pl.ds(nb, RPT_N)], aggp.at[cid, pl.ds(nb, RPT_N)])


_agg_call = pl.kernel(
    _agg_body,
    out_type=jax.ShapeDtypeStruct((2, NPAD, 16), jnp.float32),
    mesh=_MESH,
    scratch_types=[
        pltpu.VMEM((SUPE,), jnp.int32),
        pltpu.VMEM((SUPE,), jnp.int32),
        pltpu.VMEM((SUPE, 16), jnp.float32),
        pltpu.VMEM((SUPE,), jnp.int32),
        pltpu.VMEM((SUPE,), jnp.int32),
        pltpu.VMEM((TAIL,), jnp.int32),
        pltpu.VMEM((TAIL,), jnp.int32),
        pltpu.VMEM((128, 16), jnp.float32),
        pltpu.VMEM_SHARED((NPAD, 16), jnp.float32),
        pltpu.SemaphoreType.DMA,
        pltpu.SemaphoreType.DMA,
        pltpu.SemaphoreType.DMA,
    ],
    compiler_params=pltpu.CompilerParams(use_tc_tiling_on_sc=False),
)


BR = 2048
BL = BR // 8
GRID = NPAD // BR


def _lin_body(xv, bw, xl_lin):
    xl_lin[:] = jnp.dot(xv[:], bw[:], preferred_element_type=jnp.float32)


_lin_call = pl.pallas_call(
    _lin_body,
    grid=(GRID,),
    in_specs=[
        pl.BlockSpec((BL, 40), lambda i: (i, 0)),
        pl.BlockSpec((40, 128), lambda i: (0, 0)),
    ],
    out_specs=pl.BlockSpec((BL, 128), lambda i: (i, 0)),
    out_shape=jax.ShapeDtypeStruct((NL, 128), jnp.float32),
)


def _scale_body(d0, d1, xl, e8, y1, dinv_e):
    di = lax.rsqrt(1.0 + d0[:] + d1[:])
    de = jnp.dot(di, e8[:], preferred_element_type=jnp.float32)
    dinv_e[:] = de
    y1[:] = xl[:] * de


_scale_call = pl.pallas_call(
    _scale_body,
    grid=(GRID,),
    in_specs=[
        pl.BlockSpec((BL, 8), lambda i: (i, 0)),
        pl.BlockSpec((BL, 8), lambda i: (i + GRID, 0)),
        pl.BlockSpec((BL, 128), lambda i: (i, 0)),
        pl.BlockSpec((8, 128), lambda i: (0, 0)),
    ],
    out_specs=[
        pl.BlockSpec((BL, 128), lambda i: (i, 0)),
        pl.BlockSpec((BL, 128), lambda i: (i, 0)),
    ],
    out_shape=[
        jax.ShapeDtypeStruct((NL, 128), jnp.float32),
        jax.ShapeDtypeStruct((NL, 128), jnp.float32),
    ],
)


def _mid_body(a0, a1, y1, de, b1e, z2):
    i = pl.program_id(0)
    h = de[:] * (a0[:] + a1[:] + y1[:]) + b1e[:][None, :]
    h = jnp.maximum(h, 0.0)
    rows = i * BL + lax.broadcasted_iota(jnp.int32, (BL, 1), 0)
    z2[:] = jnp.where(rows < N0 // 8, de[:] * h, 0.0)


_mid_call = pl.pallas_call(
    _mid_body,
    grid=(GRID,),
    in_specs=[
        pl.BlockSpec((BL, 128), lambda i: (i, 0)),
        pl.BlockSpec((BL, 128), lambda i: (i + GRID, 0)),
        pl.BlockSpec((BL, 128), lambda i: (i, 0)),
        pl.BlockSpec((BL, 128), lambda i: (i, 0)),
        pl.BlockSpec((128,), lambda i: (0,)),
    ],
    out_specs=pl.BlockSpec((BL, 128), lambda i: (i, 0)),
    out_shape=jax.ShapeDtypeStruct((NL, 128), jnp.float32),
)


def _fin_body(a0, a1, z2, de, w2b, b2e, swp, o):
    g = de[:] * (a0[:] + a1[:] + z2[:])
    t = jnp.dot(g, w2b[:], preferred_element_type=jnp.float32)
    t = t + b2e[:][None, :]
    tsw = jnp.dot(t, swp[:], preferred_element_type=jnp.float32)
    m = jnp.maximum(t, tsw)
    s = t - m
    es = jnp.exp(s)
    essw = jnp.dot(es, swp[:], preferred_element_type=jnp.float32)
    o[:] = s - jnp.log(es + essw)


_fin_call = pl.pallas_call(
    _fin_body,
    grid=(GRID,),
    in_specs=[
        pl.BlockSpec((BL, 128), lambda i: (i, 0)),
        pl.BlockSpec((BL, 128), lambda i: (i + GRID, 0)),
        pl.BlockSpec((BL, 128), lambda i: (i, 0)),
        pl.BlockSpec((BL, 128), lambda i: (i, 0)),
        pl.BlockSpec((128, 16), lambda i: (0, 0)),
        pl.BlockSpec((16,), lambda i: (0,)),
        pl.BlockSpec((16, 16), lambda i: (0, 0)),
    ],
    out_specs=pl.BlockSpec((BL, 16), lambda i: (i, 0)),
    out_shape=jax.ShapeDtypeStruct((N0 // 8, 16), jnp.float32),
)


def kernel(x, edge_index, W1, b1, W2, b2):
    src1 = edge_index[0]
    dst1 = edge_index[1]
    xv = jnp.zeros((NL, 40), jnp.float32).at[:N0 // 8].set(
        x.astype(jnp.float32).reshape(N0 // 8, 40))
    eye8 = jnp.eye(8, dtype=jnp.float32)
    bw = jnp.kron(eye8, W1)
    e8 = jnp.kron(eye8, jnp.ones((1, 16), jnp.float32))
    w2b = jnp.kron(eye8, W2)
    swp = jnp.kron(eye8, jnp.array([[0., 1.], [1., 0.]],
                                   jnp.float32))
    b1e = jnp.tile(b1, 8)
    b2e = jnp.tile(b2, 8)

    xl = _lin_call(xv, bw)
    degp8 = _deg_call(dst1).reshape(2 * NL, 8)
    y1, dinv_e = _scale_call(degp8, degp8, xl, e8)
    a1 = _agg_call(y1.reshape(NPAD, 16), src1, dst1).reshape(2 * NL, 128)
    z2 = _mid_call(a1, a1, y1, dinv_e, b1e)
    a2 = _agg_call(z2.reshape(NPAD, 16), src1, dst1).reshape(2 * NL, 128)
    o16 = _fin_call(a2, a2, z2, dinv_e, w2b, b2e, swp)
    return o16.reshape(N0, 2)


# speedup vs baseline: 1.4928x; 1.0942x over previous
import jax
import jax.numpy as jnp
from jax import lax
from jax.experimental import pallas as pl
from jax.experimental.pallas import tpu as pltpu
from jax.experimental.pallas import tpu_sc as plsc

N0 = 100000
NPAD = 100352
NL = NPAD // 8
RPT_N = NPAD // 16
E0 = 3200000
EPT = E0 // 32
SUPE = 1024
NFULL = 97
TAIL = EPT - NFULL * SUPE

_MESH = plsc.VectorSubcoreMesh(core_axis_name="c", subcore_axis_name="s",
                               num_cores=2, num_subcores=16)


def _deg_body(dst1, degp, idxd0, idxd1, idxt, ones_v, zbuf, accd,
              semd0, semd1):
    cid = lax.axis_index("c")
    sid = lax.axis_index("s")
    wid = sid * 2 + cid
    zv = jnp.zeros((16,), jnp.float32)
    ov = jnp.ones((16,), jnp.float32)

    def fill_z(k, carry):
        zbuf[pl.ds(k * 16, 16)] = zv
        return carry
    lax.fori_loop(0, RPT_N // 16, fill_z, 0)

    def fill_o(i, carry):
        ones_v[pl.ds(i * 16, 16)] = ov
        return carry
    lax.fori_loop(0, SUPE // 16, fill_o, 0)

    nb = sid * RPT_N
    pltpu.sync_copy(zbuf, accd.at[pl.ds(nb, RPT_N)])
    plsc.subcore_barrier()

    eb = wid * EPT

    idxd = (idxd0, idxd1)
    pltpu.sync_copy(dst1.at[pl.ds(eb, SUPE)], idxd[0])

    def dpair(s, carry):
        for b in range(2):
            c = s * 2 + b
            p = b
            q = 1 - b
            sc = pltpu.async_copy(ones_v, accd.at[idxd[p]],
                                  semd0 if b == 0 else semd1, add=True)
            pltpu.sync_copy(dst1.at[pl.ds(eb + (c + 1) * SUPE, SUPE)],
                            idxd[q])
            sc.wait()
        return carry
    lax.fori_loop(0, (NFULL - 1) // 2, dpair, 0)
    sc = pltpu.async_copy(ones_v, accd.at[idxd[0]], semd0, add=True)
    pltpu.sync_copy(dst1.at[pl.ds(eb + NFULL * SUPE, TAIL)], idxt)
    sc.wait()
    pltpu.async_copy(ones_v.at[pl.ds(0, TAIL)], accd.at[idxt], semd1,
                     add=True).wait()
    plsc.subcore_barrier()
    pltpu.sync_copy(accd.at[pl.ds(nb, RPT_N)], degp.at[cid, pl.ds(nb, RPT_N)])


_deg_call = pl.kernel(
    _deg_body,
    out_type=jax.ShapeDtypeStruct((2, NPAD), jnp.float32),
    mesh=_MESH,
    scratch_types=[
        pltpu.VMEM((SUPE,), jnp.int32),
        pltpu.VMEM((SUPE,), jnp.int32),
        pltpu.VMEM((TAIL,), jnp.int32),
        pltpu.VMEM((SUPE,), jnp.float32),
        pltpu.VMEM((RPT_N,), jnp.float32),
        pltpu.VMEM_SHARED((NPAD,), jnp.float32),
        pltpu.SemaphoreType.DMA,
        pltpu.SemaphoreType.DMA,
    ],
    compiler_params=pltpu.CompilerParams(use_tc_tiling_on_sc=False),
)


def _agg_body(table, src1, dst1, aggp,
              idxs0, idxd0, rows0, idxs1, idxd1, idxst, idxdt, zbuf, acc,
              semg0, semg1, sems0):
    cid = lax.axis_index("c")
    sid = lax.axis_index("s")
    wid = sid * 2 + cid
    zv = jnp.zeros((16,), jnp.float32)

    idxs = (idxs0, idxs1)
    idxd = (idxd0, idxd1)

    def fill_z(i, carry):
        zbuf[i, :] = zv
        return carry
    lax.fori_loop(0, 128, fill_z, 0)

    nb = sid * RPT_N
    zcps = [pltpu.async_copy(zbuf, acc.at[pl.ds(nb + j * 128, 128)], semg1)
            for j in range(RPT_N // 128)]
    for cp in zcps:
        cp.wait()
    plsc.subcore_barrier()

    eb = wid * EPT

    def load_idx(c, p):
        rb = eb + c * SUPE
        pltpu.sync_copy(src1.at[pl.ds(rb, SUPE)], idxs[p])
        pltpu.sync_copy(dst1.at[pl.ds(rb, SUPE)], idxd[p])

    load_idx(0, 0)

    def pair(s, carry):
        for b in range(2):
            c = s * 2 + b
            p = b
            q = 1 - b
            pltpu.async_copy(table.at[idxs[p]], rows0, semg0).wait()
            sc = pltpu.async_copy(rows0, acc.at[idxd[p]], sems0, add=True)
            load_idx(c + 1, q)
            sc.wait()
        return carry
    lax.fori_loop(0, (NFULL - 1) // 2, pair, 0)
    pltpu.async_copy(table.at[idxs[0]], rows0, semg0).wait()
    sc = pltpu.async_copy(rows0, acc.at[idxd[0]], sems0, add=True)
    tb = eb + NFULL * SUPE
    pltpu.sync_copy(src1.at[pl.ds(tb, TAIL)], idxst)
    pltpu.sync_copy(dst1.at[pl.ds(tb, TAIL)], idxdt)
    sc.wait()
    pltpu.async_copy(table.at[idxst], rows0.at[pl.ds(0, TAIL)], semg0).wait()
    pltpu.async_copy(rows0.at[pl.ds(0, TAIL)], acc.at[idxdt], sems0,
                     add=True).wait()
    plsc.subcore_barrier()
    pltpu.sync_copy(acc.at[pl.ds(nb, RPT_N)], aggp.at[cid, pl.ds(nb, RPT_N)])


_agg_call = pl.kernel(
    _agg_body,
    out_type=jax.ShapeDtypeStruct((2, NPAD, 16), jnp.float32),
    mesh=_MESH,
    scratch_types=[
        pltpu.VMEM((SUPE,), jnp.int32),
        pltpu.VMEM((SUPE,), jnp.int32),
        pltpu.VMEM((SUPE, 16), jnp.float32),
        pltpu.VMEM((SUPE,), jnp.int32),
        pltpu.VMEM((SUPE,), jnp.int32),
        pltpu.VMEM((TAIL,), jnp.int32),
        pltpu.VMEM((TAIL,), jnp.int32),
        pltpu.VMEM((128, 16), jnp.float32),
        pltpu.VMEM_SHARED((NPAD, 16), jnp.float32),
        pltpu.SemaphoreType.DMA,
        pltpu.SemaphoreType.DMA,
        pltpu.SemaphoreType.DMA,
    ],
    compiler_params=pltpu.CompilerParams(use_tc_tiling_on_sc=False),
)


BR = 14336
BL = BR // 8
GRID = NPAD // BR


def _lin_body(xv, bw, xl_lin):
    xl_lin[:] = jnp.dot(xv[:], bw[:], preferred_element_type=jnp.float32)


_lin_call = pl.pallas_call(
    _lin_body,
    grid=(GRID,),
    in_specs=[
        pl.BlockSpec((BL, 40), lambda i: (i, 0)),
        pl.BlockSpec((40, 128), lambda i: (0, 0)),
    ],
    out_specs=pl.BlockSpec((BL, 128), lambda i: (i, 0)),
    out_shape=jax.ShapeDtypeStruct((NL, 128), jnp.float32),
)


def _scale_body(d0, d1, xl, e8, y1, dinv_e):
    di = lax.rsqrt(1.0 + d0[:] + d1[:])
    de = jnp.dot(di, e8[:], preferred_element_type=jnp.float32)
    dinv_e[:] = de
    y1[:] = xl[:] * de


_scale_call = pl.pallas_call(
    _scale_body,
    grid=(GRID,),
    in_specs=[
        pl.BlockSpec((BL, 8), lambda i: (i, 0)),
        pl.BlockSpec((BL, 8), lambda i: (i + GRID, 0)),
        pl.BlockSpec((BL, 128), lambda i: (i, 0)),
        pl.BlockSpec((8, 128), lambda i: (0, 0)),
    ],
    out_specs=[
        pl.BlockSpec((BL, 128), lambda i: (i, 0)),
        pl.BlockSpec((BL, 128), lambda i: (i, 0)),
    ],
    out_shape=[
        jax.ShapeDtypeStruct((NL, 128), jnp.float32),
        jax.ShapeDtypeStruct((NL, 128), jnp.float32),
    ],
)


def _mid_body(a0, a1, y1, de, b1e, z2):
    i = pl.program_id(0)
    h = de[:] * (a0[:] + a1[:] + y1[:]) + b1e[:][None, :]
    h = jnp.maximum(h, 0.0)
    rows = i * BL + lax.broadcasted_iota(jnp.int32, (BL, 1), 0)
    z2[:] = jnp.where(rows < N0 // 8, de[:] * h, 0.0)


_mid_call = pl.pallas_call(
    _mid_body,
    grid=(GRID,),
    in_specs=[
        pl.BlockSpec((BL, 128), lambda i: (i, 0)),
        pl.BlockSpec((BL, 128), lambda i: (i + GRID, 0)),
        pl.BlockSpec((BL, 128), lambda i: (i, 0)),
        pl.BlockSpec((BL, 128), lambda i: (i, 0)),
        pl.BlockSpec((128,), lambda i: (0,)),
    ],
    out_specs=pl.BlockSpec((BL, 128), lambda i: (i, 0)),
    out_shape=jax.ShapeDtypeStruct((NL, 128), jnp.float32),
)


def _fin_body(a0, a1, z2, de, w2b, b2e, swp, o):
    g = de[:] * (a0[:] + a1[:] + z2[:])
    t = jnp.dot(g, w2b[:], preferred_element_type=jnp.float32)
    t = t + b2e[:][None, :]
    tsw = jnp.dot(t, swp[:], preferred_element_type=jnp.float32)
    m = jnp.maximum(t, tsw)
    s = t - m
    es = jnp.exp(s)
    essw = jnp.dot(es, swp[:], preferred_element_type=jnp.float32)
    o[:] = s - jnp.log(es + essw)


_fin_call = pl.pallas_call(
    _fin_body,
    grid=(GRID,),
    in_specs=[
        pl.BlockSpec((BL, 128), lambda i: (i, 0)),
        pl.BlockSpec((BL, 128), lambda i: (i + GRID, 0)),
        pl.BlockSpec((BL, 128), lambda i: (i, 0)),
        pl.BlockSpec((BL, 128), lambda i: (i, 0)),
        pl.BlockSpec((128, 16), lambda i: (0, 0)),
        pl.BlockSpec((16,), lambda i: (0,)),
        pl.BlockSpec((16, 16), lambda i: (0, 0)),
    ],
    out_specs=pl.BlockSpec((BL, 16), lambda i: (i, 0)),
    out_shape=jax.ShapeDtypeStruct((N0 // 8, 16), jnp.float32),
)


def kernel(x, edge_index, W1, b1, W2, b2):
    src1 = edge_index[0]
    dst1 = edge_index[1]
    xv = jnp.zeros((NL, 40), jnp.float32).at[:N0 // 8].set(
        x.astype(jnp.float32).reshape(N0 // 8, 40))
    eye8 = jnp.eye(8, dtype=jnp.float32)
    bw = jnp.kron(eye8, W1)
    e8 = jnp.kron(eye8, jnp.ones((1, 16), jnp.float32))
    w2b = jnp.kron(eye8, W2)
    swp = jnp.kron(eye8, jnp.array([[0., 1.], [1., 0.]],
                                   jnp.float32))
    b1e = jnp.tile(b1, 8)
    b2e = jnp.tile(b2, 8)

    xl = _lin_call(xv, bw)
    degp8 = _deg_call(dst1).reshape(2 * NL, 8)
    y1, dinv_e = _scale_call(degp8, degp8, xl, e8)
    a1 = _agg_call(y1.reshape(NPAD, 16), src1, dst1).reshape(2 * NL, 128)
    z2 = _mid_call(a1, a1, y1, dinv_e, b1e)
    a2 = _agg_call(z2.reshape(NPAD, 16), src1, dst1).reshape(2 * NL, 128)
    o16 = _fin_call(a2, a2, z2, dinv_e, w2b, b2e, swp)
    return o16.reshape(N0, 2)
